# Initial kernel scaffold; baseline (speedup 1.0000x reference)
import jax
import jax.numpy as jnp
from jax.experimental import pallas as pl


def kernel(X, Y):
    raise NotImplementedError("write your pallas kernel here")


# speedup vs baseline: 2.7768x; 2.7768x over previous
import jax
import jax.numpy as jnp
import numpy as np
from jax.experimental import pallas as pl
from jax.experimental.pallas import tpu as pltpu

N = 1024
B = 8
NUM_RINGS = 512
EPS = 1e-8
ROWS_1A = 512
COLS_1B = 512
ROWS_RING = 8

_VMEM = pltpu.CompilerParams(
    dimension_semantics=("arbitrary",),
    vmem_limit_bytes=60 * 1024 * 1024,
)
_VMEM2 = pltpu.CompilerParams(
    dimension_semantics=("arbitrary", "arbitrary"),
    vmem_limit_bytes=60 * 1024 * 1024,
)


def _dft_mats():
    u = np.arange(N, dtype=np.int64)
    m = (u[:, None] * u[None, :]) % N
    theta = (2.0 * np.pi / N) * m.astype(np.float64)
    return np.cos(theta).astype(np.float32), (-np.sin(theta)).astype(np.float32)


def _hilo_np(a32):
    import ml_dtypes
    hi = a32.astype(ml_dtypes.bfloat16)
    lo = (a32 - hi.astype(np.float32)).astype(ml_dtypes.bfloat16)
    return hi, lo


_DR, _DI = _dft_mats()
_DRH, _DRL = _hilo_np(_DR)
_DIH, _DIL = _hilo_np(_DI)


def _hilo(a):
    hi = a.astype(jnp.bfloat16)
    lo = (a - hi.astype(jnp.float32)).astype(jnp.bfloat16)
    return hi, lo


def _mm3(ah, al, bh, bl):
    f32 = jnp.float32
    return (jnp.dot(ah, bh, preferred_element_type=f32)
            + jnp.dot(ah, bl, preferred_element_type=f32)
            + jnp.dot(al, bh, preferred_element_type=f32))


def _stage1_kernel(x_ref, y_ref, drh, drl, dih, dil, tr_ref, ti_ref):
    xh, xl = _hilo(x_ref[...])
    yh, yl = _hilo(y_ref[...])
    tr_ref[...] = _mm3(xh, xl, drh[...], drl[...]) - _mm3(yh, yl, dih[...], dil[...])
    ti_ref[...] = _mm3(xh, xl, dih[...], dil[...]) + _mm3(yh, yl, drh[...], drl[...])


def _stage2_kernel(tr_ref, ti_ref, drh, drl, dih, dil, fr_ref, fi_ref):
    trh, trl = _hilo(tr_ref[...])
    tih, til = _hilo(ti_ref[...])
    fr_ref[...] = _mm3(drh[...], drl[...], trh, trl) - _mm3(dih[...], dil[...], tih, til)
    fi_ref[...] = _mm3(drh[...], drl[...], tih, til) + _mm3(dih[...], dil[...], trh, trl)


def _ring_kernel(fr_ref, fi_ref, br_ref, bi_ref, out_ref):
    i = pl.program_id(0)
    fr, fi = fr_ref[...], fi_ref[...]
    br, bi = br_ref[...], bi_ref[...]
    col = jax.lax.broadcasted_iota(jnp.int32, (1, N), 1)
    fx = jnp.where(col < N // 2, col, col - N).astype(jnp.float32)
    fx2 = fx * fx
    ringv = jax.lax.broadcasted_iota(
        jnp.int32, (NUM_RINGS, 1), 0).astype(jnp.float32)
    rsq_lo = ringv * ringv
    rsq_hi = (ringv + 1.0) * (ringv + 1.0)
    nt = (((1,), (1,)), ((), ()))

    acc = jnp.zeros((NUM_RINGS, 3 * B), jnp.float32)
    for r in range(ROWS_RING):
        ar, ai = fr[:, r, :], fi[:, r, :]
        brr, bri = br[:, r, :], bi[:, r, :]
        a = ar * ar + ai * ai
        q1 = ar * brr - ai * bri
        q2 = ar * bri + ai * brr
        data = jnp.concatenate([a, q1, q2], axis=0)

        row = i * ROWS_RING + r
        fy = jnp.where(row < N // 2, row, row - N).astype(jnp.float32)
        r2 = fy * fy + fx2
        onehot = ((r2 >= rsq_lo) & (r2 < rsq_hi)).astype(jnp.bfloat16)

        dh, dl = _hilo(data)
        acc = acc + jax.lax.dot_general(
            onehot, dh, nt, preferred_element_type=jnp.float32)
        acc = acc + jax.lax.dot_general(
            onehot, dl, nt, preferred_element_type=jnp.float32)

    @pl.when(i == 0)
    def _():
        out_ref[...] = jnp.zeros_like(out_ref)

    out_ref[...] += acc


def _dft_all(x2, y2):
    d_spec = pl.BlockSpec((N, N), lambda *_: (0, 0))
    dmats = (jnp.asarray(_DRH), jnp.asarray(_DRL),
             jnp.asarray(_DIH), jnp.asarray(_DIL))

    row_spec = pl.BlockSpec((ROWS_1A, N), lambda i: (i, 0))
    tr, ti = pl.pallas_call(
        _stage1_kernel,
        grid=(B * N // ROWS_1A,),
        in_specs=[row_spec, row_spec, d_spec, d_spec, d_spec, d_spec],
        out_specs=[row_spec, row_spec],
        out_shape=[jax.ShapeDtypeStruct((B * N, N), jnp.float32)] * 2,
        compiler_params=_VMEM,
        name="frc_dft_rows",
    )(x2, y2, *dmats)

    t_spec = pl.BlockSpec((N, COLS_1B), lambda b, c: (b, c))
    fr, fi = pl.pallas_call(
        _stage2_kernel,
        grid=(B, N // COLS_1B),
        in_specs=[t_spec, t_spec, d_spec, d_spec, d_spec, d_spec],
        out_specs=[t_spec, t_spec],
        out_shape=[jax.ShapeDtypeStruct((B * N, N), jnp.float32)] * 2,
        compiler_params=_VMEM2,
        name="frc_dft_cols",
    )(tr, ti, *dmats)
    return fr, fi


def _ring_sums(fr, fi, br, bi):
    blk = pl.BlockSpec((B, ROWS_RING, N), lambda i: (0, i, 0))
    return pl.pallas_call(
        _ring_kernel,
        grid=(N // ROWS_RING,),
        in_specs=[blk, blk, blk, blk],
        out_specs=pl.BlockSpec((NUM_RINGS, 3 * B), lambda i: (0, 0)),
        out_shape=jax.ShapeDtypeStruct((NUM_RINGS, 3 * B), jnp.float32),
        compiler_params=_VMEM,
        name="frc_rings",
    )(fr, fi, br, bi)


@jax.jit
def kernel(X, Y):
    x2 = X.reshape(B * N, N)
    y2 = Y.reshape(B * N, N)
    fr2, fi2 = _dft_all(x2, y2)
    fr = fr2.reshape(B, N, N)
    fi = fi2.reshape(B, N, N)
    br = jnp.roll(jnp.flip(fr, axis=(1, 2)), shift=(1, 1), axis=(1, 2))
    bi = jnp.roll(jnp.flip(fi, axis=(1, 2)), shift=(1, 1), axis=(1, 2))
    sums = _ring_sums(fr, fi, br, bi)
    s0 = 2.0 * sums[:, 0 * B:1 * B]
    s1r = sums[:, 1 * B:2 * B]
    s2i = sums[:, 2 * B:3 * B]
    cs = 0.5 * s2i
    p1 = 0.25 * (s0 + 2.0 * s1r)
    p2 = 0.25 * (s0 - 2.0 * s1r)
    frc = cs / jnp.sqrt(p1 * p2 + EPS)
    return jnp.mean(frc)


# speedup vs baseline: 3.4852x; 1.2551x over previous
import jax
import jax.numpy as jnp
import numpy as np
from jax.experimental import pallas as pl
from jax.experimental.pallas import tpu as pltpu

N = 1024
B = 8
NUM_RINGS = 512
EPS = 1e-8
ROWS_1A = 512
COLS_1B = 512
ROWS_RING = 8

_VMEM = pltpu.CompilerParams(
    dimension_semantics=("arbitrary",),
    vmem_limit_bytes=60 * 1024 * 1024,
)
_VMEM2 = pltpu.CompilerParams(
    dimension_semantics=("arbitrary", "arbitrary"),
    vmem_limit_bytes=60 * 1024 * 1024,
)


def _dft_mats():
    u = np.arange(N, dtype=np.int64)
    m = (u[:, None] * u[None, :]) % N
    theta = (2.0 * np.pi / N) * m.astype(np.float64)
    return np.cos(theta).astype(np.float32), (-np.sin(theta)).astype(np.float32)


def _hilo_np(a32):
    import ml_dtypes
    hi = a32.astype(ml_dtypes.bfloat16)
    lo = (a32 - hi.astype(np.float32)).astype(ml_dtypes.bfloat16)
    return hi, lo


_DR, _DI = _dft_mats()
_DRH, _DRL = _hilo_np(_DR)
_DIH, _DIL = _hilo_np(_DI)


def _hilo(a):
    hi = a.astype(jnp.bfloat16)
    lo = (a - hi.astype(jnp.float32)).astype(jnp.bfloat16)
    return hi, lo


def _mm3(ah, al, bh, bl):
    f32 = jnp.float32
    return (jnp.dot(ah, bh, preferred_element_type=f32)
            + jnp.dot(ah, bl, preferred_element_type=f32)
            + jnp.dot(al, bh, preferred_element_type=f32))


def _stage1_kernel(x_ref, y_ref, drh, drl, dih, dil, tr_ref, ti_ref):
    xh, xl = _hilo(x_ref[...])
    yh, yl = _hilo(y_ref[...])
    tr_ref[...] = _mm3(xh, xl, drh[...], drl[...]) - _mm3(yh, yl, dih[...], dil[...])
    ti_ref[...] = _mm3(xh, xl, dih[...], dil[...]) + _mm3(yh, yl, drh[...], drl[...])


def _stage2_kernel(tr_ref, ti_ref, drh, drl, dih, dil, fr_ref, fi_ref):
    trh, trl = _hilo(tr_ref[...])
    tih, til = _hilo(ti_ref[...])
    fr_ref[...] = _mm3(drh[...], drl[...], trh, trl) - _mm3(dih[...], dil[...], tih, til)
    fi_ref[...] = _mm3(drh[...], drl[...], tih, til) + _mm3(dih[...], dil[...], trh, trl)


def _ring_kernel(fr_ref, fi_ref, br_ref, bi_ref, out_ref):
    i = pl.program_id(0)
    fr, fi = fr_ref[...], fi_ref[...]
    br, bi = br_ref[...], bi_ref[...]
    col = jax.lax.broadcasted_iota(jnp.int32, (1, N), 1)
    fx = jnp.where(col < N // 2, col, col - N).astype(jnp.float32)
    fx2 = fx * fx
    ringv = jax.lax.broadcasted_iota(
        jnp.int32, (NUM_RINGS, 1), 0).astype(jnp.float32)
    rsq_lo = ringv * ringv
    rsq_hi = (ringv + 1.0) * (ringv + 1.0)
    nt = (((1,), (1,)), ((), ()))

    acc = jnp.zeros((NUM_RINGS, 3 * B), jnp.float32)
    for r in range(ROWS_RING):
        ar, ai = fr[:, r, :], fi[:, r, :]
        brr, bri = br[:, r, :], bi[:, r, :]
        a = ar * ar + ai * ai
        q1 = ar * brr - ai * bri
        q2 = ar * bri + ai * brr
        data = jnp.concatenate([a, q1, q2], axis=0)
        w = jnp.where((i == 0) & (r == 0), 1.0, 2.0).astype(jnp.float32)
        data = data * w

        row = i * ROWS_RING + r
        fy = jnp.float32(1.0) * row
        r2 = fy * fy + fx2
        onehot = ((r2 >= rsq_lo) & (r2 < rsq_hi)).astype(jnp.bfloat16)

        dh, dl = _hilo(data)
        acc = acc + jax.lax.dot_general(
            onehot, dh, nt, preferred_element_type=jnp.float32)
        acc = acc + jax.lax.dot_general(
            onehot, dl, nt, preferred_element_type=jnp.float32)

    @pl.when(i == 0)
    def _():
        out_ref[...] = jnp.zeros_like(out_ref)

    out_ref[...] += acc


def _dft_all(x2, y2):
    d_spec = pl.BlockSpec((N, N), lambda *_: (0, 0))
    dmats = (jnp.asarray(_DRH), jnp.asarray(_DRL),
             jnp.asarray(_DIH), jnp.asarray(_DIL))

    row_spec = pl.BlockSpec((ROWS_1A, N), lambda i: (i, 0))
    tr, ti = pl.pallas_call(
        _stage1_kernel,
        grid=(B * N // ROWS_1A,),
        in_specs=[row_spec, row_spec, d_spec, d_spec, d_spec, d_spec],
        out_specs=[row_spec, row_spec],
        out_shape=[jax.ShapeDtypeStruct((B * N, N), jnp.float32)] * 2,
        compiler_params=_VMEM,
        name="frc_dft_rows",
    )(x2, y2, *dmats)

    t_spec = pl.BlockSpec((N, COLS_1B), lambda b, c: (b, c))
    fr, fi = pl.pallas_call(
        _stage2_kernel,
        grid=(B, N // COLS_1B),
        in_specs=[t_spec, t_spec, d_spec, d_spec, d_spec, d_spec],
        out_specs=[t_spec, t_spec],
        out_shape=[jax.ShapeDtypeStruct((B * N, N), jnp.float32)] * 2,
        compiler_params=_VMEM2,
        name="frc_dft_cols",
    )(tr, ti, *dmats)
    return fr, fi


def _ring_sums(fr, fi, br, bi):
    blk = pl.BlockSpec((B, ROWS_RING, N), lambda i: (0, i, 0))
    return pl.pallas_call(
        _ring_kernel,
        grid=(N // 2 // ROWS_RING,),
        in_specs=[blk, blk, blk, blk],
        out_specs=pl.BlockSpec((NUM_RINGS, 3 * B), lambda i: (0, 0)),
        out_shape=jax.ShapeDtypeStruct((NUM_RINGS, 3 * B), jnp.float32),
        compiler_params=_VMEM,
        name="frc_rings",
    )(fr, fi, br, bi)


@jax.jit
def kernel(X, Y):
    x2 = X.reshape(B * N, N)
    y2 = Y.reshape(B * N, N)
    fr2, fi2 = _dft_all(x2, y2)
    fr = fr2.reshape(B, N, N)
    fi = fi2.reshape(B, N, N)
    br = jnp.roll(jnp.flip(fr, axis=(1, 2)), shift=(1, 1), axis=(1, 2))
    bi = jnp.roll(jnp.flip(fi, axis=(1, 2)), shift=(1, 1), axis=(1, 2))
    half = N // 2
    sums = _ring_sums(fr[:, :half], fi[:, :half],
                      br[:, :half], bi[:, :half])
    s0 = 2.0 * sums[:, 0 * B:1 * B]
    s1r = sums[:, 1 * B:2 * B]
    s2i = sums[:, 2 * B:3 * B]
    cs = 0.5 * s2i
    p1 = 0.25 * (s0 + 2.0 * s1r)
    p2 = 0.25 * (s0 - 2.0 * s1r)
    frc = cs / jnp.sqrt(p1 * p2 + EPS)
    return jnp.mean(frc)
